# baseline (speedup 1.0000x reference)
import functools

import jax
import jax.numpy as jnp
from jax import lax
from jax.experimental import pallas as pl
from jax.experimental.pallas import tpu as pltpu
from jax.experimental.pallas import tpu_sc as plsc

D_MODEL = 2048
SCALE = float(D_MODEL) ** 0.5
NC, NS, L = 2, 16, 16
NW = NC * NS
B_TOTAL = 4 * 4096
B_PER_W = B_TOTAL // NW
CG = 16
CS = 8
N_G = B_PER_W // CG
N_ROUNDS = (N_G - 2) // 2


@functools.cache
def _make_emb():
    mesh = plsc.VectorSubcoreMesh(
        core_axis_name="c", subcore_axis_name="s",
        num_cores=NC, num_subcores=NS)

    @functools.partial(
        pl.kernel,
        out_type=jax.ShapeDtypeStruct((B_TOTAL, D_MODEL), jnp.float32),
        mesh=mesh,
        scratch_types=(
            [pltpu.VMEM((B_PER_W,), jnp.int32)]
            + [pltpu.VMEM((CG, D_MODEL), jnp.float32)] * 2
            + [pltpu.VMEM((CS, D_MODEL), jnp.float32)] * 2
            + [pltpu.SemaphoreType.DMA] * 4
        ),
    )
    def emb(idx_hbm, table_hbm, out_hbm, idx_v,
            gb0, gb1, sb0, sb1, sem_g0, sem_g1, sem_s0, sem_s1):
        wid = lax.axis_index("s") * NC + lax.axis_index("c")
        base = wid * B_PER_W
        pltpu.sync_copy(idx_hbm.at[pl.ds(base, B_PER_W)], idx_v)

        gbufs = ((gb0, sem_g0), (gb1, sem_g1))
        sbufs = ((sb0, sem_s0), (sb1, sem_s1))

        def gather(gb, sem, G):
            return pltpu.make_async_copy(
                table_hbm.at[idx_v.at[pl.ds(G * CG, CG)]], gb, sem)

        def scatter(sb, sem, h):
            return pltpu.make_async_copy(
                sb, out_hbm.at[pl.ds(base + h * CS, CS)], sem)

        def scale_half(gb, half, sb):
            for r in range(CS):
                @plsc.parallel_loop(0, D_MODEL // L, unroll=8)
                def _(i):
                    sl = pl.ds(i * L, L)
                    sb[r, sl] = gb[half * CS + r, sl] * SCALE

        gather(gb0, sem_g0, 0).start()
        gather(gb1, sem_g1, 1).start()

        gather(gb0, sem_g0, 0).wait()
        scale_half(gb0, 0, sb0)
        scatter(sb0, sem_s0, 0).start()
        scale_half(gb0, 1, sb1)
        scatter(sb1, sem_s1, 1).start()
        gather(gb0, sem_g0, 2).start()

        gather(gb1, sem_g1, 0).wait()
        scatter(sb0, sem_s0, 0).wait()
        scale_half(gb1, 0, sb0)
        scatter(sb0, sem_s0, 2).start()
        scatter(sb1, sem_s1, 0).wait()
        scale_half(gb1, 1, sb1)
        scatter(sb1, sem_s1, 3).start()
        gather(gb1, sem_g1, 3).start()

        def round_body(p, carry):
            for a in range(2):
                G = 2 * p + 2 + a
                gb, sg = gbufs[a]
                gather(gb, sg, 0).wait()
                for half in range(2):
                    sb, ss = sbufs[half]
                    scatter(sb, ss, 0).wait()
                    scale_half(gb, half, sb)
                    scatter(sb, ss, 2 * G + half).start()
                gather(gb, sg, (G + 2) & (N_G - 1)).start()
            return carry

        lax.fori_loop(0, N_ROUNDS, round_body, None)

        gather(gb0, sem_g0, 0).wait()
        gather(gb1, sem_g1, 0).wait()
        scatter(sb0, sem_s0, 0).wait()
        scatter(sb1, sem_s1, 0).wait()

    return emb


def kernel(x, embedding_table):
    b, s = x.shape
    x_flat = x.reshape(-1).astype(jnp.int32)
    out = _make_emb()(x_flat, embedding_table)
    return out.reshape(b, s, D_MODEL)


# speedup vs baseline: 1.0429x; 1.0429x over previous
import functools

import jax
import jax.numpy as jnp
from jax import lax
from jax.experimental import pallas as pl
from jax.experimental.pallas import tpu as pltpu
from jax.experimental.pallas import tpu_sc as plsc

D_MODEL = 2048
SCALE = float(D_MODEL) ** 0.5
NC, NS, L = 2, 16, 16
NW = NC * NS
B_TOTAL = 4 * 4096
B_PER_W = B_TOTAL // NW
C = 8
N_CHUNKS = B_PER_W // C
SLOTS = 3
N_ROUNDS = (N_CHUNKS - 1) // SLOTS


@functools.cache
def _make_emb():
    mesh = plsc.VectorSubcoreMesh(
        core_axis_name="c", subcore_axis_name="s",
        num_cores=NC, num_subcores=NS)

    vmem_row_buf = pltpu.VMEM((C, D_MODEL), jnp.float32)

    @functools.partial(
        pl.kernel,
        out_type=jax.ShapeDtypeStruct((B_TOTAL, D_MODEL), jnp.float32),
        mesh=mesh,
        scratch_types=(
            [pltpu.VMEM((B_PER_W,), jnp.int32)]
            + [vmem_row_buf] * (2 * SLOTS)
            + [pltpu.SemaphoreType.DMA] * (2 * SLOTS)
        ),
    )
    def emb(idx_hbm, table_hbm, out_hbm, idx_v,
            g0, g1, g2, s0, s1, s2,
            sem_g0, sem_g1, sem_g2, sem_s0, sem_s1, sem_s2):
        wid = lax.axis_index("s") * NC + lax.axis_index("c")
        base = wid * B_PER_W
        pltpu.sync_copy(idx_hbm.at[pl.ds(base, B_PER_W)], idx_v)

        gbufs = ((g0, sem_g0), (g1, sem_g1), (g2, sem_g2))
        sbufs = ((s0, sem_s0), (s1, sem_s1), (s2, sem_s2))

        def gather(gb, sem, g):
            return pltpu.make_async_copy(
                table_hbm.at[idx_v.at[pl.ds(g * C, C)]], gb, sem)

        def scatter(sb, sem, g):
            return pltpu.make_async_copy(
                sb, out_hbm.at[pl.ds(base + g * C, C)], sem)

        def scale(gb, sb):
            for r in range(C):
                @plsc.parallel_loop(0, D_MODEL // L, unroll=8)
                def _(i):
                    sl = pl.ds(i * L, L)
                    sb[r, sl] = gb[r, sl] * SCALE

        for s in range(SLOTS):
            gather(gbufs[s][0], gbufs[s][1], s).start()

        def round_body(p, carry):
            for s in range(SLOTS):
                g = SLOTS * p + s
                gb, sg = gbufs[s]
                sb, ss = sbufs[s]
                gather(gb, sg, 0).wait()
                @pl.when(p > 0)
                def _():
                    scatter(sb, ss, 0).wait()
                scale(gb, sb)
                scatter(sb, ss, g).start()
                @pl.when(g + SLOTS < N_CHUNKS)
                def _():
                    gather(gb, sg, g + SLOTS).start()
            return carry

        lax.fori_loop(0, N_ROUNDS, round_body, None)

        gather(g0, sem_g0, 0).wait()
        scatter(s0, sem_s0, 0).wait()
        scale(g0, s0)
        scatter(s0, sem_s0, N_CHUNKS - 1).start()

        for s in range(SLOTS):
            scatter(sbufs[s][0], sbufs[s][1], 0).wait()

    return emb


def kernel(x, embedding_table):
    b, s = x.shape
    x_flat = x.reshape(-1).astype(jnp.int32)
    out = _make_emb()(x_flat, embedding_table)
    return out.reshape(b, s, D_MODEL)
